# Initial kernel scaffold; baseline (speedup 1.0000x reference)
import jax
import jax.numpy as jnp
from jax.experimental import pallas as pl


def kernel(feats, W1, W2a, g2a, b2a, W2b, W3a, g3a, b3a, W3b, g3b, b3b, W3c, gf, bf, nbr_pool, nbr5, cells):
    raise NotImplementedError("write your pallas kernel here")


# speedup vs baseline: 11.6359x; 11.6359x over previous
import jax
import jax.numpy as jnp
from jax import lax
from jax.experimental import pallas as pl
from jax.experimental.pallas import tpu as pltpu, tpu_sc as plsc

_N = 65536
_C = 32
_HG, _WG = 176, 608
_G = _HG * _WG
_GP = 107520
_CHUNK = _GP // 16
_TH = 8
_NT = _HG // _TH
_HP, _WP = 192, 612


def _sc_body(mcC):
    mc = mcC * 128

    def body(fpad_hbm, nbr_hbm, gidx_hbm, zeros_hbm, ones_hbm,
             outg, outm, fpad_v, nbr_v, gidx_v, f0c, ones_v, zbuf,
             shared_g, shared_m):
        cid = lax.axis_index("c")
        sid = lax.axis_index("s")
        wid = cid * 16 + sid
        off = sid * _CHUNK

        pltpu.sync_copy(zeros_hbm, zbuf)
        pltpu.sync_copy(zbuf, shared_g.at[pl.ds(off, _CHUNK)])
        pltpu.sync_copy(zbuf, shared_m.at[pl.ds(off, _CHUNK)])
        pltpu.sync_copy(fpad_hbm, fpad_v)
        pltpu.sync_copy(nbr_hbm.at[wid], nbr_v)
        pltpu.sync_copy(gidx_hbm.at[wid], gidx_v)
        pltpu.sync_copy(ones_hbm, ones_v)
        plsc.subcore_barrier()

        def chunk(c, carry):
            cbase = pl.multiple_of(c * 128, 128)
            for i in range(8):
                acc = plsc.load_gather(
                    fpad_v, [nbr_v[pl.ds(cbase + i * 16, 16)]])
                for k in range(1, 9):
                    acc = jnp.maximum(acc, plsc.load_gather(
                        fpad_v, [nbr_v[pl.ds(cbase + k * mc + i * 16, 16)]]))
                f0c[pl.ds(i * 16, 16)] = acc
            pltpu.sync_copy(f0c, shared_g.at[gidx_v.at[c]], add=True)
            pltpu.sync_copy(ones_v, shared_m.at[gidx_v.at[c]], add=True)
            return carry

        lax.fori_loop(0, mcC, chunk, 0)
        plsc.subcore_barrier()

        base = cid * _GP + off
        pltpu.sync_copy(shared_g.at[pl.ds(off, _CHUNK)], zbuf)
        pltpu.sync_copy(zbuf, outg.at[pl.ds(base, _CHUNK)])
        pltpu.sync_copy(shared_m.at[pl.ds(off, _CHUNK)], zbuf)
        pltpu.sync_copy(zbuf, outm.at[pl.ds(base, _CHUNK)])

    return body


def _sc_densify(fpad, nbr, gidx, mcC):
    k = pl.kernel(
        _sc_body(mcC),
        out_type=(jax.ShapeDtypeStruct((2 * _GP,), jnp.float32),
                  jax.ShapeDtypeStruct((2 * _GP,), jnp.float32)),
        mesh=plsc.VectorSubcoreMesh(core_axis_name="c", subcore_axis_name="s"),
        compiler_params=pltpu.CompilerParams(needs_layout_passes=False),
        scratch_types=[
            pltpu.VMEM((_N + 8,), jnp.float32),
            pltpu.VMEM((9 * mcC * 128,), jnp.int32),
            pltpu.VMEM((mcC, 128), jnp.int32),
            pltpu.VMEM((128,), jnp.float32),
            pltpu.VMEM((128,), jnp.float32),
            pltpu.VMEM((_CHUNK,), jnp.float32),
            pltpu.VMEM_SHARED((_GP,), jnp.float32),
            pltpu.VMEM_SHARED((_GP,), jnp.float32),
        ],
    )
    zeros_src = jnp.zeros((_CHUNK,), jnp.float32)
    ones_src = jnp.ones((128,), jnp.float32)
    return k(fpad, nbr, gidx, zeros_src, ones_src)


_BIG = 1e30
_ROWS = _TH * _WG


def _k1_body(t_ref, m8_ref, w_ref, o_ref, st_ref):
    i = pl.program_id(0)
    t = t_ref[...]
    acc = jnp.dot(t, w_ref[...], preferred_element_type=jnp.float32)
    mcol = jnp.max(m8_ref[...], axis=1, keepdims=True)

    @pl.when(i == 0)
    def _():
        st_ref[...] = jnp.zeros_like(st_ref)

    om = acc * mcol
    st_ref[0:1, :] += jnp.sum(om, axis=0, keepdims=True)
    st_ref[1:2, :] += jnp.sum(om * acc, axis=0, keepdims=True)
    o_ref[...] = acc * mcol + (mcol - 1.0) * _BIG


def _conv1(taps, m8, wcat):
    r1 = _G // 44
    return pl.pallas_call(
        _k1_body,
        grid=(44,),
        in_specs=[
            pl.BlockSpec((r1, 25), lambda i: (i, 0)),
            pl.BlockSpec((r1, 8), lambda i: (i, 0)),
            pl.BlockSpec((25, 160), lambda i: (0, 0)),
        ],
        out_specs=[
            pl.BlockSpec((r1, 160), lambda i: (i, 0)),
            pl.BlockSpec((8, 160), lambda i: (0, 0)),
        ],
        out_shape=[
            jax.ShapeDtypeStruct((_G, 160), jnp.float32),
            jax.ShapeDtypeStruct((8, 160), jnp.float32),
        ],
    )(taps, m8, wcat)


def _mk_conv_body(cout, with_mask, with_add, with_stats, encode):
    def body(*refs):
        xA, xB, w_ref, s_ref, t_ref = refs[:5]
        idx = 5
        if with_mask:
            m8 = refs[idx]; idx += 1
        if with_add:
            add2 = refs[idx]; idx += 1
        o = refs[idx]; idx += 1
        if with_stats:
            st = refs[idx]; idx += 1

        i = pl.program_id(0)
        win = jnp.concatenate([xA[...], xB[...]], axis=0)
        h = jnp.maximum(win * s_ref[...] + t_ref[...], 0.0)
        acc = jnp.zeros((_ROWS, cout), jnp.float32)
        k = 0
        for dy in range(5):
            for dx in range(5):
                hs = h[dy:dy + _TH, dx:dx + _WG, :].reshape(_ROWS, 64)
                acc = acc + jnp.dot(hs, w_ref[k],
                                    preferred_element_type=jnp.float32)
                k += 1
        if with_add:
            acc = acc + add2[...]
        if with_mask:
            mcol = jnp.max(m8[...], axis=1, keepdims=True)
        if with_stats:
            @pl.when(i == 0)
            def _():
                st[...] = jnp.zeros_like(st)
            om = acc * mcol
            st[0:1, :] += jnp.sum(om, axis=0, keepdims=True)
            st[1:2, :] += jnp.sum(om * acc, axis=0, keepdims=True)
        if encode:
            acc = acc * mcol + (mcol - 1.0) * _BIG
        o[...] = acc

    return body


def _conv(xp, w, s, t, cout, m8=None, add=None, stats=False, encode=False):
    in_specs = [
        pl.BlockSpec((_TH, _WP, 64), lambda i: (i, 0, 0)),
        pl.BlockSpec((_TH, _WP, 64), lambda i: (i + 1, 0, 0)),
        pl.BlockSpec((25, 64, cout), lambda i: (0, 0, 0)),
        pl.BlockSpec((1, 1, 64), lambda i: (0, 0, 0)),
        pl.BlockSpec((1, 1, 64), lambda i: (0, 0, 0)),
    ]
    args = [xp, xp, w, s.reshape(1, 1, 64), t.reshape(1, 1, 64)]
    if m8 is not None:
        in_specs.append(pl.BlockSpec((_ROWS, 8), lambda i: (i, 0)))
        args.append(m8)
    if add is not None:
        in_specs.append(pl.BlockSpec((_ROWS, cout), lambda i: (i, 0)))
        args.append(add)
    out_specs = [pl.BlockSpec((_ROWS, cout), lambda i: (i, 0))]
    out_shape = [jax.ShapeDtypeStruct((_G, cout), jnp.float32)]
    if stats:
        out_specs.append(pl.BlockSpec((8, cout), lambda i: (0, 0)))
        out_shape.append(jax.ShapeDtypeStruct((8, cout), jnp.float32))
    res = pl.pallas_call(
        _mk_conv_body(cout, m8 is not None, add is not None, stats, encode),
        grid=(_NT,),
        in_specs=in_specs,
        out_specs=out_specs,
        out_shape=out_shape,
    )(*args)
    return res if stats else res[0]


def _k4_body(z_ref, s_ref, t_ref, y_ref):
    y_ref[...] = jnp.maximum(z_ref[...] * s_ref[...] + t_ref[...], 0.0)


def _finish(qz, s, t):
    return pl.pallas_call(
        _k4_body,
        grid=(_NT,),
        in_specs=[
            pl.BlockSpec((_ROWS, _C), lambda i: (i, 0)),
            pl.BlockSpec((1, _C), lambda i: (0, 0)),
            pl.BlockSpec((1, _C), lambda i: (0, 0)),
        ],
        out_specs=pl.BlockSpec((_ROWS, _C), lambda i: (i, 0)),
        out_shape=jax.ShapeDtypeStruct((_G, _C), jnp.float32),
    )(qz, s.reshape(1, _C), t.reshape(1, _C))


def _pad_q(x2d, c):
    x = x2d.reshape(_HG, _WG, c)
    return jnp.pad(x, ((2, _HP - _HG - 2), (2, _WP - _WG - 2), (0, 0)),
                   constant_values=-1e30)


def _fold_bn(st, g, b, m_count):
    mean = st[0] / m_count
    var = st[1] / m_count - mean * mean
    inv = g / jnp.sqrt(var + 1e-5)
    return inv.astype(jnp.float32), (b - mean * inv).astype(jnp.float32)


def kernel(feats, W1, W2a, g2a, b2a, W2b, W3a, g3a, b3a, W3b, g3b, b3b, W3c,
           gf, bf, nbr_pool, nbr5, cells):
    M = nbr_pool.shape[0]
    mcC = -(-M // 4096)
    Mp = mcC * 4096
    pad_rows = Mp - M

    fpad = jnp.concatenate(
        [feats.reshape(-1).astype(jnp.float32),
         jnp.full((8,), -1e30, jnp.float32)])
    nbrT = jnp.concatenate(
        [nbr_pool.astype(jnp.int32),
         jnp.full((pad_rows, 9), _N, jnp.int32)], axis=0)
    nbrT = nbrT.T.reshape(9, 32, mcC * 128).transpose(1, 0, 2)
    nbrT = nbrT.reshape(32, 9 * mcC * 128)
    gidx = cells[:, 0].astype(jnp.int32) * _WG + cells[:, 1].astype(jnp.int32)
    gidx = jnp.concatenate([gidx, jnp.full((pad_rows,), _G, jnp.int32)])
    gidx = gidx.reshape(32, mcC, 128)

    outg, outm = _sc_densify(fpad, nbrT, gidx, mcC)
    g0 = (outg[:_GP] + outg[_GP:])[:_G].reshape(_HG, _WG)
    mk = (outm[:_GP] + outm[_GP:])[:_G].reshape(_HG, _WG)

    g0p = jnp.pad(g0, ((2, _HP - _HG - 2), (2, _WP - _WG - 2)))
    taps = jnp.stack(
        [g0p[dy:dy + _HG, dx:dx + _WG].reshape(-1)
         for dy in range(5) for dx in range(5)], axis=-1)
    m8 = jnp.tile(mk.reshape(_G, 1), (1, 8))

    wcat = jnp.concatenate(
        [W1.reshape(25, _C), W2a.reshape(25, 64), W3a.reshape(25, 64)], axis=1)

    q160, st160 = _conv1(taps, m8, wcat)
    x1q = q160[:, :_C]
    qa2 = q160[:, _C:_C + 64]
    qa3 = q160[:, _C + 64:]

    s2, t2 = _fold_bn(st160[:, _C:_C + 64], g2a, b2a, M)
    s3, t3 = _fold_bn(st160[:, _C + 64:], g3a, b3a, M)

    p12 = _conv(_pad_q(qa2, 64), W2b, s2, t2, _C, add=x1q)
    qb3, stb3 = _conv(_pad_q(qa3, 64), W3b, s3, t3, 64, m8=m8, stats=True,
                      encode=True)
    s3b, t3b = _fold_bn(stb3, g3b, b3b, M)
    qz, stz = _conv(_pad_q(qb3, 64), W3c, s3b, t3b, _C, m8=m8, add=p12,
                    stats=True, encode=True)
    sf, tf = _fold_bn(stz, gf, bf, M)

    y = _finish(qz, sf, tf)
    return jnp.transpose(y.reshape(_HG, _WG, _C), (2, 0, 1))[None]


# speedup vs baseline: 14.2979x; 1.2288x over previous
import jax
import jax.numpy as jnp
from jax import lax
from jax.experimental import pallas as pl
from jax.experimental.pallas import tpu as pltpu, tpu_sc as plsc

_N = 65536
_C = 32
_HG, _WG = 176, 608
_G = _HG * _WG
_GP = 107520
_CHUNK = _GP // 16
_TH = 8
_NT = _HG // _TH
_HP, _WP = 192, 612


def _sc_body(mcC):
    mc = mcC * 128

    def body(fpad_hbm, nbr_hbm, gidx_hbm, zeros_hbm, ones_hbm,
             outg, outm, fpad_v, nbr_v, gidx_v, f0c, ones_v, zbuf,
             shared_g, shared_m):
        cid = lax.axis_index("c")
        sid = lax.axis_index("s")
        wid = cid * 16 + sid
        off = sid * _CHUNK

        pltpu.sync_copy(zeros_hbm, zbuf)
        pltpu.sync_copy(zbuf, shared_g.at[pl.ds(off, _CHUNK)])
        pltpu.sync_copy(zbuf, shared_m.at[pl.ds(off, _CHUNK)])
        pltpu.sync_copy(fpad_hbm, fpad_v)
        pltpu.sync_copy(nbr_hbm.at[wid], nbr_v)
        pltpu.sync_copy(gidx_hbm.at[wid], gidx_v)
        pltpu.sync_copy(ones_hbm, ones_v)
        plsc.subcore_barrier()

        def chunk(c, carry):
            cbase = pl.multiple_of(c * 128, 128)
            for i in range(8):
                acc = plsc.load_gather(
                    fpad_v, [nbr_v[pl.ds(cbase + i * 16, 16)]])
                for k in range(1, 9):
                    acc = jnp.maximum(acc, plsc.load_gather(
                        fpad_v, [nbr_v[pl.ds(cbase + k * mc + i * 16, 16)]]))
                f0c[pl.ds(i * 16, 16)] = acc
            pltpu.sync_copy(f0c, shared_g.at[gidx_v.at[c]], add=True)
            pltpu.sync_copy(ones_v, shared_m.at[gidx_v.at[c]], add=True)
            return carry

        lax.fori_loop(0, mcC, chunk, 0)
        plsc.subcore_barrier()

        base = cid * _GP + off
        pltpu.sync_copy(shared_g.at[pl.ds(off, _CHUNK)], zbuf)
        pltpu.sync_copy(zbuf, outg.at[pl.ds(base, _CHUNK)])
        pltpu.sync_copy(shared_m.at[pl.ds(off, _CHUNK)], zbuf)
        pltpu.sync_copy(zbuf, outm.at[pl.ds(base, _CHUNK)])

    return body


def _sc_densify(fpad, nbr, gidx, mcC):
    k = pl.kernel(
        _sc_body(mcC),
        out_type=(jax.ShapeDtypeStruct((2 * _GP,), jnp.float32),
                  jax.ShapeDtypeStruct((2 * _GP,), jnp.float32)),
        mesh=plsc.VectorSubcoreMesh(core_axis_name="c", subcore_axis_name="s"),
        compiler_params=pltpu.CompilerParams(needs_layout_passes=False),
        scratch_types=[
            pltpu.VMEM((_N + 8,), jnp.float32),
            pltpu.VMEM((9 * mcC * 128,), jnp.int32),
            pltpu.VMEM((mcC, 128), jnp.int32),
            pltpu.VMEM((128,), jnp.float32),
            pltpu.VMEM((128,), jnp.float32),
            pltpu.VMEM((_CHUNK,), jnp.float32),
            pltpu.VMEM_SHARED((_GP,), jnp.float32),
            pltpu.VMEM_SHARED((_GP,), jnp.float32),
        ],
    )
    zeros_src = jnp.zeros((_CHUNK,), jnp.float32)
    ones_src = jnp.ones((128,), jnp.float32)
    return k(fpad, nbr, gidx, zeros_src, ones_src)


_BIG = 1e30
_ROWS = _TH * _WG


def _k1_body(t_ref, m8_ref, w_ref, o_ref, st_ref):
    i = pl.program_id(0)
    t = t_ref[...]
    acc = jnp.dot(t, w_ref[...], preferred_element_type=jnp.float32)
    mcol = jnp.max(m8_ref[...], axis=1, keepdims=True)

    @pl.when(i == 0)
    def _():
        st_ref[...] = jnp.zeros_like(st_ref)

    om = acc * mcol
    st_ref[0:1, :] += jnp.sum(om, axis=0, keepdims=True)
    st_ref[1:2, :] += jnp.sum(om * acc, axis=0, keepdims=True)
    o_ref[...] = acc * mcol + (mcol - 1.0) * _BIG


def _conv1(taps, m8, wcat):
    r1 = _G // 44
    return pl.pallas_call(
        _k1_body,
        grid=(44,),
        in_specs=[
            pl.BlockSpec((r1, 25), lambda i: (i, 0)),
            pl.BlockSpec((r1, 8), lambda i: (i, 0)),
            pl.BlockSpec((25, 160), lambda i: (0, 0)),
        ],
        out_specs=[
            pl.BlockSpec((r1, 160), lambda i: (i, 0)),
            pl.BlockSpec((8, 160), lambda i: (0, 0)),
        ],
        out_shape=[
            jax.ShapeDtypeStruct((_G, 160), jnp.float32),
            jax.ShapeDtypeStruct((8, 160), jnp.float32),
        ],
    )(taps, m8, wcat)


def _mk_conv_body(cout, with_mask, with_add, with_stats, encode):
    def body(*refs):
        xA, xB, w_ref, s_ref, t_ref = refs[:5]
        idx = 5
        if with_mask:
            m8 = refs[idx]; idx += 1
        if with_add:
            add2 = refs[idx]; idx += 1
        o = refs[idx]; idx += 1
        if with_stats:
            st = refs[idx]; idx += 1

        i = pl.program_id(0)
        win = jnp.concatenate([xA[...], xB[...]], axis=0)
        h = jnp.maximum(win * s_ref[...] + t_ref[...], 0.0)
        acc = jnp.zeros((_ROWS, cout), jnp.float32)
        for dy in range(5):
            hs = jnp.concatenate(
                [h[dy:dy + _TH, dx:dx + _WG, :] for dx in range(5)],
                axis=2).reshape(_ROWS, 320)
            acc = acc + jnp.dot(hs, w_ref[dy],
                                preferred_element_type=jnp.float32)
        if with_add:
            acc = acc + add2[...]
        if with_mask:
            mcol = jnp.max(m8[...], axis=1, keepdims=True)
        if with_stats:
            @pl.when(i == 0)
            def _():
                st[...] = jnp.zeros_like(st)
            om = acc * mcol
            st[0:1, :] += jnp.sum(om, axis=0, keepdims=True)
            st[1:2, :] += jnp.sum(om * acc, axis=0, keepdims=True)
        if encode:
            acc = acc * mcol + (mcol - 1.0) * _BIG
        o[...] = acc

    return body


def _conv(xp, w, s, t, cout, m8=None, add=None, stats=False, encode=False):
    in_specs = [
        pl.BlockSpec((_TH, _WP, 64), lambda i: (i, 0, 0)),
        pl.BlockSpec((_TH, _WP, 64), lambda i: (i + 1, 0, 0)),
        pl.BlockSpec((5, 320, cout), lambda i: (0, 0, 0)),
        pl.BlockSpec((1, 1, 64), lambda i: (0, 0, 0)),
        pl.BlockSpec((1, 1, 64), lambda i: (0, 0, 0)),
    ]
    args = [xp, xp, w.reshape(5, 320, cout),
            s.reshape(1, 1, 64), t.reshape(1, 1, 64)]
    if m8 is not None:
        in_specs.append(pl.BlockSpec((_ROWS, 8), lambda i: (i, 0)))
        args.append(m8)
    if add is not None:
        in_specs.append(pl.BlockSpec((_ROWS, cout), lambda i: (i, 0)))
        args.append(add)
    out_specs = [pl.BlockSpec((_ROWS, cout), lambda i: (i, 0))]
    out_shape = [jax.ShapeDtypeStruct((_G, cout), jnp.float32)]
    if stats:
        out_specs.append(pl.BlockSpec((8, cout), lambda i: (0, 0)))
        out_shape.append(jax.ShapeDtypeStruct((8, cout), jnp.float32))
    res = pl.pallas_call(
        _mk_conv_body(cout, m8 is not None, add is not None, stats, encode),
        grid=(_NT,),
        in_specs=in_specs,
        out_specs=out_specs,
        out_shape=out_shape,
    )(*args)
    return res if stats else res[0]


def _k4_body(z_ref, s_ref, t_ref, y_ref):
    y_ref[...] = jnp.maximum(z_ref[...] * s_ref[...] + t_ref[...], 0.0)


def _finish(qz, s, t):
    return pl.pallas_call(
        _k4_body,
        grid=(_NT,),
        in_specs=[
            pl.BlockSpec((_ROWS, _C), lambda i: (i, 0)),
            pl.BlockSpec((1, _C), lambda i: (0, 0)),
            pl.BlockSpec((1, _C), lambda i: (0, 0)),
        ],
        out_specs=pl.BlockSpec((_ROWS, _C), lambda i: (i, 0)),
        out_shape=jax.ShapeDtypeStruct((_G, _C), jnp.float32),
    )(qz, s.reshape(1, _C), t.reshape(1, _C))


def _pad_q(x2d, c):
    x = x2d.reshape(_HG, _WG, c)
    return jnp.pad(x, ((2, _HP - _HG - 2), (2, _WP - _WG - 2), (0, 0)),
                   constant_values=-1e30)


def _fold_bn(st, g, b, m_count):
    mean = st[0] / m_count
    var = st[1] / m_count - mean * mean
    inv = g / jnp.sqrt(var + 1e-5)
    return inv.astype(jnp.float32), (b - mean * inv).astype(jnp.float32)


def kernel(feats, W1, W2a, g2a, b2a, W2b, W3a, g3a, b3a, W3b, g3b, b3b, W3c,
           gf, bf, nbr_pool, nbr5, cells):
    M = nbr_pool.shape[0]
    mcC = -(-M // 4096)
    Mp = mcC * 4096
    pad_rows = Mp - M

    fpad = jnp.concatenate(
        [feats.reshape(-1).astype(jnp.float32),
         jnp.full((8,), -1e30, jnp.float32)])
    nbrT = jnp.concatenate(
        [nbr_pool.astype(jnp.int32),
         jnp.full((pad_rows, 9), _N, jnp.int32)], axis=0)
    nbrT = nbrT.T.reshape(9, 32, mcC * 128).transpose(1, 0, 2)
    nbrT = nbrT.reshape(32, 9 * mcC * 128)
    gidx = cells[:, 0].astype(jnp.int32) * _WG + cells[:, 1].astype(jnp.int32)
    gidx = jnp.concatenate([gidx, jnp.full((pad_rows,), _G, jnp.int32)])
    gidx = gidx.reshape(32, mcC, 128)

    outg, outm = _sc_densify(fpad, nbrT, gidx, mcC)
    g0 = (outg[:_GP] + outg[_GP:])[:_G].reshape(_HG, _WG)
    mk = (outm[:_GP] + outm[_GP:])[:_G].reshape(_HG, _WG)

    g0p = jnp.pad(g0, ((2, _HP - _HG - 2), (2, _WP - _WG - 2)))
    taps = jnp.stack(
        [g0p[dy:dy + _HG, dx:dx + _WG].reshape(-1)
         for dy in range(5) for dx in range(5)], axis=-1)
    m8 = jnp.tile(mk.reshape(_G, 1), (1, 8))

    wcat = jnp.concatenate(
        [W1.reshape(25, _C), W2a.reshape(25, 64), W3a.reshape(25, 64)], axis=1)

    q160, st160 = _conv1(taps, m8, wcat)
    x1q = q160[:, :_C]
    qa2 = q160[:, _C:_C + 64]
    qa3 = q160[:, _C + 64:]

    s2, t2 = _fold_bn(st160[:, _C:_C + 64], g2a, b2a, M)
    s3, t3 = _fold_bn(st160[:, _C + 64:], g3a, b3a, M)

    p12 = _conv(_pad_q(qa2, 64), W2b, s2, t2, _C, add=x1q)
    qb3, stb3 = _conv(_pad_q(qa3, 64), W3b, s3, t3, 64, m8=m8, stats=True,
                      encode=True)
    s3b, t3b = _fold_bn(stb3, g3b, b3b, M)
    qz, stz = _conv(_pad_q(qb3, 64), W3c, s3b, t3b, _C, m8=m8, add=p12,
                    stats=True, encode=True)
    sf, tf = _fold_bn(stz, gf, bf, M)

    y = _finish(qz, sf, tf)
    return jnp.transpose(y.reshape(_HG, _WG, _C), (2, 0, 1))[None]
